# baseline (speedup 1.0000x reference)
import functools

import jax
import jax.numpy as jnp
from jax import lax
from jax.experimental import pallas as pl
from jax.experimental.pallas import tpu as pltpu
from jax.experimental.pallas import tpu_sc as plsc

D = 64
SPLITS = (104, 96)
NB = 2


def _build(B, S, V):
    info = plsc.get_sparse_core_info()
    NC, NS = info.num_cores, info.num_subcores
    NW = NC * NS
    assert S == sum(SPLITS) and B % (NW * NB) == 0
    bat_per_w = B // NW
    n_chunks = bat_per_w // NB
    assert n_chunks % 2 == 0
    G = n_chunks // 2
    CHUNKI = NB * S

    mesh = plsc.VectorSubcoreMesh(core_axis_name="c", subcore_axis_name="s")

    @functools.partial(
        pl.kernel,
        mesh=mesh,
        out_type=jax.ShapeDtypeStruct((B, S, D), jnp.float32),
        compiler_params=pltpu.CompilerParams(use_tc_tiling_on_sc=False),
        scratch_types=[
            pltpu.VMEM((2, CHUNKI), jnp.int32),
            pltpu.VMEM((2, NB, S, D), jnp.float32),
            pltpu.VMEM_SHARED((V, D), jnp.float32),
            pltpu.SemaphoreType.DMA,
            pltpu.SemaphoreType.DMA,
            pltpu.SemaphoreType.DMA,
            pltpu.SemaphoreType.DMA,
            pltpu.SemaphoreType.DMA,
        ],
    )
    def k(table_hbm, idx_hbm, out_hbm, idx_v, rows_v, table_sh, gat_sem,
          idx_sem0, idx_sem1, out_sem0, out_sem1):
        wid = lax.axis_index("s") * NC + lax.axis_index("c")
        wbat = wid * bat_per_w

        @pl.when(lax.axis_index("s") == 0)
        def _stage():
            pltpu.sync_copy(table_hbm, table_sh)
        plsc.subcore_barrier()

        idx_sems = (idx_sem0, idx_sem1)
        out_sems = (out_sem0, out_sem1)

        def idx_copy(i, b):
            return pltpu.make_async_copy(
                idx_hbm.at[pl.ds((wbat + i * NB) * S, CHUNKI)],
                idx_v.at[b],
                idx_sems[b],
            )

        def out_copy(i, b):
            return pltpu.make_async_copy(
                rows_v.at[b], out_hbm.at[pl.ds(wbat + i * NB, NB)], out_sems[b]
            )

        idx_copy(0, 0).start()
        idx_copy(1, 1).start()

        def body(g, _):
            for b in (0, 1):
                i = 2 * g + b
                idx_copy(i, b).wait()
                @pl.when(g > 0)
                def _drain_out():
                    out_copy(i, b).wait()
                handles = []
                for q in range(NB):
                    off = 0
                    for w in SPLITS:
                        handles.append(pltpu.async_copy(
                            table_sh.at[idx_v.at[b, pl.ds(q * S + off, w)]],
                            rows_v.at[b, q, pl.ds(off, w)],
                            gat_sem,
                        ))
                        off += w
                for h in handles:
                    h.wait()
                out_copy(i, b).start()
                @pl.when(g < G - 1)
                def _prefetch():
                    idx_copy(i + 2, b).start()
            return 0

        lax.fori_loop(0, G, body, 0)

        for b in (0, 1):
            out_copy(0, b).wait()

    return k


def kernel(positions, table):
    B, S = positions.shape
    V, d = table.shape
    idx = positions.reshape(B * S).astype(jnp.int32)
    return _build(B, S, V)(table, idx)


# speedup vs baseline: 1.6675x; 1.6675x over previous
import functools

import jax
import jax.numpy as jnp
from jax import lax
from jax.experimental import pallas as pl
from jax.experimental.pallas import tpu as pltpu
from jax.experimental.pallas import tpu_sc as plsc

D = 64
DP = 128
IW = 128
K = 2
CHUNK = K * IW


def _build(N, V):
    info = plsc.get_sparse_core_info()
    NC, NS = info.num_cores, info.num_subcores
    NW = NC * NS
    assert N % (NW * CHUNK) == 0
    b_per_w = N // NW
    n_chunks = b_per_w // CHUNK
    assert n_chunks % 2 == 0
    G = n_chunks // 2

    mesh = plsc.VectorSubcoreMesh(core_axis_name="c", subcore_axis_name="s")

    @functools.partial(
        pl.kernel,
        mesh=mesh,
        out_type=jax.ShapeDtypeStruct((N, DP), jnp.float32),
        compiler_params=pltpu.CompilerParams(use_tc_tiling_on_sc=False),
        scratch_types=[
            pltpu.VMEM((2, K, IW), jnp.int32),
            pltpu.VMEM((2, CHUNK, DP), jnp.float32),
            pltpu.VMEM_SHARED((V, DP), jnp.float32),
            pltpu.SemaphoreType.DMA,
            pltpu.SemaphoreType.DMA,
            pltpu.SemaphoreType.DMA,
            pltpu.SemaphoreType.DMA,
            pltpu.SemaphoreType.DMA,
        ],
    )
    def k(table_hbm, idx_hbm, out_hbm, idx_v, rows_v, table_sh, gat_sem,
          idx_sem0, idx_sem1, out_sem0, out_sem1):
        wid = lax.axis_index("s") * NC + lax.axis_index("c")
        base = wid * b_per_w
        rbase = base // IW

        @pl.when(lax.axis_index("s") == 0)
        def _stage():
            pltpu.sync_copy(table_hbm, table_sh)
        plsc.subcore_barrier()

        idx_sems = (idx_sem0, idx_sem1)
        out_sems = (out_sem0, out_sem1)

        def idx_copy(i, b):
            return pltpu.make_async_copy(
                idx_hbm.at[pl.ds(rbase + i * K, K)], idx_v.at[b], idx_sems[b]
            )

        def out_copy(i, b):
            return pltpu.make_async_copy(
                rows_v.at[b], out_hbm.at[pl.ds(base + i * CHUNK, CHUNK)],
                out_sems[b],
            )

        idx_copy(0, 0).start()
        idx_copy(1, 1).start()

        def body(g, _):
            for b in (0, 1):
                i = 2 * g + b
                idx_copy(i, b).wait()
                @pl.when(g > 0)
                def _drain_out():
                    out_copy(i, b).wait()
                handles = []
                for j in range(K):
                    handles.append(pltpu.async_copy(
                        table_sh.at[idx_v.at[b, j]],
                        rows_v.at[b, pl.ds(j * IW, IW)],
                        gat_sem,
                    ))
                for h in handles:
                    h.wait()
                out_copy(i, b).start()
                @pl.when(g < G - 1)
                def _prefetch():
                    idx_copy(i + 2, b).start()
            return 0

        lax.fori_loop(0, G, body, 0)

        for b in (0, 1):
            out_copy(0, b).wait()

    return k


def kernel(positions, table):
    B, S = positions.shape
    V, d = table.shape
    N = B * S
    idx = positions.reshape(N // IW, IW).astype(jnp.int32)
    table_pad = jnp.pad(table, ((0, 0), (0, DP - d)))
    out = _build(N, V)(table_pad, idx)
    return out[:, :d].reshape(B, S, d)


# speedup vs baseline: 2.1151x; 1.2684x over previous
import functools

import jax
import jax.numpy as jnp
from jax import lax
from jax.experimental import pallas as pl
from jax.experimental.pallas import tpu as pltpu
from jax.experimental.pallas import tpu_sc as plsc

D = 64
DP = 128
IW = 128
K = 2
CHUNK = K * IW


def _build(N, V):
    info = plsc.get_sparse_core_info()
    NC, NS = info.num_cores, info.num_subcores
    NW = NC * NS
    assert N % (NW * CHUNK) == 0
    b_per_w = N // NW
    n_chunks = b_per_w // CHUNK
    assert n_chunks % 2 == 0
    G = n_chunks // 2

    mesh = plsc.VectorSubcoreMesh(core_axis_name="c", subcore_axis_name="s")

    @functools.partial(
        pl.kernel,
        mesh=mesh,
        out_type=jax.ShapeDtypeStruct((N, DP), jnp.float32),
        compiler_params=pltpu.CompilerParams(use_tc_tiling_on_sc=False),
        scratch_types=[
            pltpu.VMEM((2, K, IW), jnp.int32),
            pltpu.VMEM((2, CHUNK, D), jnp.float32),
            pltpu.VMEM_SHARED((V, D), jnp.float32),
            pltpu.SemaphoreType.DMA,
            pltpu.SemaphoreType.DMA,
            pltpu.SemaphoreType.DMA,
            pltpu.SemaphoreType.DMA,
            pltpu.SemaphoreType.DMA,
        ],
    )
    def k(table_hbm, idx_hbm, out_hbm, idx_v, rows_v, table_sh, gat_sem,
          idx_sem0, idx_sem1, out_sem0, out_sem1):
        wid = lax.axis_index("s") * NC + lax.axis_index("c")
        base = wid * b_per_w
        rbase = base // IW

        @pl.when(lax.axis_index("s") == 0)
        def _stage():
            pltpu.sync_copy(table_hbm, table_sh)
        plsc.subcore_barrier()

        idx_sems = (idx_sem0, idx_sem1)
        out_sems = (out_sem0, out_sem1)

        def idx_copy(i, b):
            return pltpu.make_async_copy(
                idx_hbm.at[pl.ds(rbase + i * K, K)], idx_v.at[b], idx_sems[b]
            )

        def out_copy(i, b):
            return pltpu.make_async_copy(
                rows_v.at[b],
                out_hbm.at[pl.ds(base + i * CHUNK, CHUNK), pl.ds(0, D)],
                out_sems[b],
            )

        idx_copy(0, 0).start()
        idx_copy(1, 1).start()

        def body(g, _):
            for b in (0, 1):
                i = 2 * g + b
                idx_copy(i, b).wait()
                @pl.when(g > 0)
                def _drain_out():
                    out_copy(i, b).wait()
                handles = []
                for j in range(K):
                    handles.append(pltpu.async_copy(
                        table_sh.at[idx_v.at[b, j]],
                        rows_v.at[b, pl.ds(j * IW, IW)],
                        gat_sem,
                    ))
                for h in handles:
                    h.wait()
                out_copy(i, b).start()
                @pl.when(g < G - 1)
                def _prefetch():
                    idx_copy(i + 2, b).start()
            return 0

        lax.fori_loop(0, G, body, 0)

        for b in (0, 1):
            out_copy(0, b).wait()

    return k


def kernel(positions, table):
    B, S = positions.shape
    V, d = table.shape
    N = B * S
    idx = positions.reshape(N // IW, IW).astype(jnp.int32)
    out = _build(N, V)(table, idx)
    return out[:, :d].reshape(B, S, d)


# speedup vs baseline: 2.1587x; 1.0206x over previous
import functools

import jax
import jax.numpy as jnp
from jax import lax
from jax.experimental import pallas as pl
from jax.experimental.pallas import tpu as pltpu
from jax.experimental.pallas import tpu_sc as plsc

D = 64
DP = 128
IW = 128
K = 4
CHUNK = K * IW


def _build(N, V):
    info = plsc.get_sparse_core_info()
    NC, NS = info.num_cores, info.num_subcores
    NW = NC * NS
    assert N % (NW * CHUNK) == 0
    b_per_w = N // NW
    n_chunks = b_per_w // CHUNK
    assert n_chunks % 2 == 0
    G = n_chunks // 2

    mesh = plsc.VectorSubcoreMesh(core_axis_name="c", subcore_axis_name="s")

    @functools.partial(
        pl.kernel,
        mesh=mesh,
        out_type=jax.ShapeDtypeStruct((N, DP), jnp.float32),
        compiler_params=pltpu.CompilerParams(use_tc_tiling_on_sc=False),
        scratch_types=[
            pltpu.VMEM((2, K, IW), jnp.int32),
            pltpu.VMEM((2, CHUNK, D), jnp.float32),
            pltpu.VMEM_SHARED((V, D), jnp.float32),
            pltpu.SemaphoreType.DMA,
            pltpu.SemaphoreType.DMA,
            pltpu.SemaphoreType.DMA,
            pltpu.SemaphoreType.DMA,
            pltpu.SemaphoreType.DMA,
        ],
    )
    def k(table_hbm, idx_hbm, out_hbm, idx_v, rows_v, table_sh, gat_sem,
          idx_sem0, idx_sem1, out_sem0, out_sem1):
        wid = lax.axis_index("s") * NC + lax.axis_index("c")
        base = wid * b_per_w
        rbase = base // IW

        @pl.when(lax.axis_index("s") == 0)
        def _stage():
            pltpu.sync_copy(table_hbm, table_sh)
        plsc.subcore_barrier()

        idx_sems = (idx_sem0, idx_sem1)
        out_sems = (out_sem0, out_sem1)

        def idx_copy(i, b):
            return pltpu.make_async_copy(
                idx_hbm.at[pl.ds(rbase + i * K, K)], idx_v.at[b], idx_sems[b]
            )

        def out_copy(i, b):
            return pltpu.make_async_copy(
                rows_v.at[b],
                out_hbm.at[pl.ds(base + i * CHUNK, CHUNK), pl.ds(0, D)],
                out_sems[b],
            )

        idx_copy(0, 0).start()
        idx_copy(1, 1).start()

        def body(g, _):
            for b in (0, 1):
                i = 2 * g + b
                idx_copy(i, b).wait()
                @pl.when(g > 0)
                def _drain_out():
                    out_copy(i, b).wait()
                handles = []
                for j in range(K):
                    handles.append(pltpu.async_copy(
                        table_sh.at[idx_v.at[b, j]],
                        rows_v.at[b, pl.ds(j * IW, IW)],
                        gat_sem,
                    ))
                for h in handles:
                    h.wait()
                out_copy(i, b).start()
                @pl.when(g < G - 1)
                def _prefetch():
                    idx_copy(i + 2, b).start()
            return 0

        lax.fori_loop(0, G, body, 0)

        for b in (0, 1):
            out_copy(0, b).wait()

    return k


def kernel(positions, table):
    B, S = positions.shape
    V, d = table.shape
    N = B * S
    idx = positions.reshape(N // IW, IW).astype(jnp.int32)
    out = _build(N, V)(table, idx)
    return out[:, :d].reshape(B, S, d)


# speedup vs baseline: 2.1592x; 1.0002x over previous
import functools

import jax
import jax.numpy as jnp
from jax import lax
from jax.experimental import pallas as pl
from jax.experimental.pallas import tpu as pltpu
from jax.experimental.pallas import tpu_sc as plsc

D = 64
DP = 128
IW = 128
K = 5
CHUNK = K * IW


def _build(N, V):
    info = plsc.get_sparse_core_info()
    NC, NS = info.num_cores, info.num_subcores
    NW = NC * NS
    assert N % (NW * CHUNK) == 0
    b_per_w = N // NW
    n_chunks = b_per_w // CHUNK
    assert n_chunks % 2 == 0
    G = n_chunks // 2

    mesh = plsc.VectorSubcoreMesh(core_axis_name="c", subcore_axis_name="s")

    @functools.partial(
        pl.kernel,
        mesh=mesh,
        out_type=jax.ShapeDtypeStruct((N, DP), jnp.float32),
        compiler_params=pltpu.CompilerParams(use_tc_tiling_on_sc=False),
        scratch_types=[
            pltpu.VMEM((2, K, IW), jnp.int32),
            pltpu.VMEM((2, CHUNK, D), jnp.float32),
            pltpu.VMEM_SHARED((V, D), jnp.float32),
            pltpu.SemaphoreType.DMA,
            pltpu.SemaphoreType.DMA,
            pltpu.SemaphoreType.DMA,
            pltpu.SemaphoreType.DMA,
            pltpu.SemaphoreType.DMA,
        ],
    )
    def k(table_hbm, idx_hbm, out_hbm, idx_v, rows_v, table_sh, gat_sem,
          idx_sem0, idx_sem1, out_sem0, out_sem1):
        wid = lax.axis_index("s") * NC + lax.axis_index("c")
        base = wid * b_per_w
        rbase = base // IW

        @pl.when(lax.axis_index("s") == 0)
        def _stage():
            pltpu.sync_copy(table_hbm, table_sh)
        plsc.subcore_barrier()

        idx_sems = (idx_sem0, idx_sem1)
        out_sems = (out_sem0, out_sem1)

        def idx_copy(i, b):
            return pltpu.make_async_copy(
                idx_hbm.at[pl.ds(rbase + i * K, K)], idx_v.at[b], idx_sems[b]
            )

        def out_copy(i, b):
            return pltpu.make_async_copy(
                rows_v.at[b],
                out_hbm.at[pl.ds(base + i * CHUNK, CHUNK), pl.ds(0, D)],
                out_sems[b],
            )

        idx_copy(0, 0).start()
        idx_copy(1, 1).start()

        def body(g, _):
            for b in (0, 1):
                i = 2 * g + b
                idx_copy(i, b).wait()
                @pl.when(g > 0)
                def _drain_out():
                    out_copy(i, b).wait()
                handles = []
                for j in range(K):
                    handles.append(pltpu.async_copy(
                        table_sh.at[idx_v.at[b, j]],
                        rows_v.at[b, pl.ds(j * IW, IW)],
                        gat_sem,
                    ))
                for h in handles:
                    h.wait()
                out_copy(i, b).start()
                @pl.when(g < G - 1)
                def _prefetch():
                    idx_copy(i + 2, b).start()
            return 0

        lax.fori_loop(0, G, body, 0)

        for b in (0, 1):
            out_copy(0, b).wait()

    return k


def kernel(positions, table):
    B, S = positions.shape
    V, d = table.shape
    N = B * S
    idx = positions.reshape(N // IW, IW).astype(jnp.int32)
    out = _build(N, V)(table, idx)
    return out[:, :d].reshape(B, S, d)
